# baseline (speedup 1.0000x reference)
import functools

import jax
import jax.numpy as jnp
from jax import lax
from jax.experimental import pallas as pl
from jax.experimental.pallas import tpu as pltpu
from jax.experimental.pallas import tpu_sc as plsc

EMBED_DIM = 32
CHUNK = 128


@functools.lru_cache(maxsize=None)
def _build_detile(nu: int, nb: int):
    info = plsc.get_sparse_core_info()
    NC, NS = info.num_cores, info.num_subcores
    NW = NC * NS
    qu = nu // 128
    qb = nb // 128
    u_rows = (qu + 1) * 32
    b_rows = (qb + 1) * 32

    mesh = plsc.VectorSubcoreMesh(core_axis_name="c", subcore_axis_name="s")

    @functools.partial(
        pl.kernel,
        mesh=mesh,
        compiler_params=pltpu.CompilerParams(needs_layout_passes=False),
        out_type=(jax.ShapeDtypeStruct((u_rows, 128), jnp.float32),
                  jax.ShapeDtypeStruct((b_rows, 128), jnp.float32)),
        scratch_types=[
            pltpu.VMEM((EMBED_DIM, 129), jnp.float32),
            pltpu.VMEM((EMBED_DIM, 129), jnp.float32),
            pltpu.VMEM((EMBED_DIM, 128), jnp.float32),
            pltpu.VMEM((EMBED_DIM, 128), jnp.float32),
            pltpu.SemaphoreType.DMA,
            pltpu.SemaphoreType.DMA,
            pltpu.SemaphoreType.DMA,
        ],
    )
    def kfn(utabT, btabT, utailT, btailT, u128, b128,
            cvmA, cvmB, tvmA, tvmB, ldsem, stsemA, stsemB):
        wid = lax.axis_index("s") * NC + lax.axis_index("c")
        iota16 = lax.iota(jnp.int32, 16)
        iota16b = iota16 + 16

        def issue(src, src_q, cvm):
            col0 = pl.multiple_of(src_q * 128, 128)
            return [pltpu.async_copy(
                src.at[pl.ds(8 * p, 8), pl.ds(col0, 128)],
                cvm.at[pl.ds(8 * p, 8), pl.ds(0, 128)], ldsem)
                for p in range(4)]

        def transpose(cvm, tvm):
            def tr_body(rr, carry):
                rv = jnp.broadcast_to(rr, (16,)).astype(jnp.int32)
                g0 = plsc.load_gather(cvm, [iota16, rv])
                g1 = plsc.load_gather(cvm, [iota16b, rv])
                row = rr // 4
                col = (rr % 4) * 32
                tvm[row, pl.ds(col, 16)] = g0
                tvm[row, pl.ds(col + 16, 16)] = g1
                return carry

            lax.fori_loop(0, 128, tr_body, 0, unroll=4)

        def wait_write(tvm, stsem):
            pltpu.make_async_copy(tvm, u128.at[pl.ds(0, 32)], stsem).wait()

        def make_pair_body(src, dst, limit, first_user):
            def pair_body(k, carry):
                qA = wid + (2 * k) * NW
                qB = wid + (2 * k + 1) * NW

                @pl.when(qA < limit)
                def _():
                    loads = issue(src, qA, cvmA)
                    if first_user:
                        @pl.when(k > 0)
                        def _():
                            wait_write(tvmA, stsemA)
                    else:
                        wait_write(tvmA, stsemA)
                    for cc in loads:
                        cc.wait()
                    transpose(cvmA, tvmA)
                    row0 = pl.multiple_of(qA * 32, 32)
                    pltpu.async_copy(tvmA, dst.at[pl.ds(row0, 32)], stsemA)

                @pl.when(qB < limit)
                def _():
                    loads = issue(src, qB, cvmB)
                    if first_user:
                        @pl.when(k > 0)
                        def _():
                            wait_write(tvmB, stsemB)
                    else:
                        wait_write(tvmB, stsemB)
                    for cc in loads:
                        cc.wait()
                    transpose(cvmB, tvmB)
                    row0 = pl.multiple_of(qB * 32, 32)
                    pltpu.async_copy(tvmB, dst.at[pl.ds(row0, 32)], stsemB)

                return carry
            return pair_body

        lax.fori_loop(0, (qu + 2 * NW - 1) // (2 * NW),
                      make_pair_body(utabT, u128, qu, True), 0)
        lax.fori_loop(0, (qb + 2 * NW - 1) // (2 * NW),
                      make_pair_body(btabT, b128, qb, False), 0)

        @pl.when(wid == 0)
        def _():
            loads = issue(utailT, 0, cvmA)
            wait_write(tvmA, stsemA)
            for cc in loads:
                cc.wait()
            transpose(cvmA, tvmA)
            pltpu.async_copy(tvmA, u128.at[pl.ds(qu * 32, 32)], stsemA)

        @pl.when(wid == 1)
        def _():
            loads = issue(btailT, 0, cvmB)
            wait_write(tvmB, stsemB)
            for cc in loads:
                cc.wait()
            transpose(cvmB, tvmB)
            pltpu.async_copy(tvmB, b128.at[pl.ds(qb * 32, 32)], stsemB)

        wait_write(tvmA, stsemA)
        wait_write(tvmB, stsemB)

    return kfn


@functools.lru_cache(maxsize=None)
def _build_gather(B: int):
    info = plsc.get_sparse_core_info()
    NC, NS = info.num_cores, info.num_subcores
    NW = NC * NS
    b_per_w = B // NW
    n_chunks = b_per_w // CHUNK

    mesh = plsc.VectorSubcoreMesh(core_axis_name="c", subcore_axis_name="s")

    @functools.partial(
        pl.kernel,
        mesh=mesh,
        compiler_params=pltpu.CompilerParams(needs_layout_passes=False),
        out_type=jax.ShapeDtypeStruct((B,), jnp.float32),
        scratch_types=[
            pltpu.VMEM((b_per_w,), jnp.int32),
            pltpu.VMEM((b_per_w,), jnp.int32),
            pltpu.VMEM((b_per_w,), jnp.int32),
            pltpu.VMEM((b_per_w,), jnp.int32),
            pltpu.VMEM((CHUNK, 128), jnp.float32),
            pltpu.VMEM((CHUNK, 128), jnp.float32),
            pltpu.VMEM((EMBED_DIM,), jnp.float32),
            pltpu.VMEM((16,), jnp.float32),
            pltpu.VMEM((b_per_w,), jnp.float32),
            pltpu.VMEM((17 * 16,), jnp.float32),
            pltpu.SemaphoreType.DMA,
        ],
    )
    def kfn(uid_hbm, bid_hbm, utab_hbm, btab_hbm, w_hbm, b_hbm, out_hbm,
            uid_v, bid_v, uq_v, bq_v, ubuf_v, bbuf_v, w_v, b_v, out_v, tr_v,
            gsem):
        wid = lax.axis_index("s") * NC + lax.axis_index("c")
        base = wid * b_per_w

        pltpu.sync_copy(uid_hbm.at[pl.ds(base, b_per_w)], uid_v)
        pltpu.sync_copy(bid_hbm.at[pl.ds(base, b_per_w)], bid_v)
        pltpu.sync_copy(w_hbm, w_v)
        pltpu.sync_copy(b_hbm, b_v)

        def split_body(g, carry):
            sl = pl.ds(g * 16, 16)
            for src, qdst in ((uid_v, uq_v), (bid_v, bq_v)):
                v = src[sl]
                qdst[sl] = lax.shift_right_logical(v, 2)
                src[sl] = lax.shift_left(v & 3, 5)
            return carry

        lax.fori_loop(0, b_per_w // 16, split_body, 0)

        w0 = w_v[pl.ds(0, 16)]
        w1 = w_v[pl.ds(16, 16)]
        fcb = b_v[pl.ds(0, 16)][0]
        col_base = lax.iota(jnp.int32, 16) * 17

        for c in range(n_chunks):
            sl = pl.ds(c * CHUNK, CHUNK)
            cu = pltpu.async_copy(utab_hbm.at[uq_v.at[sl]], ubuf_v, gsem)
            cb = pltpu.async_copy(btab_hbm.at[bq_v.at[sl]], bbuf_v, gsem)
            cu.wait()
            cb.wait()

            def group_body(g, carry, c=c):
                r0 = g * 16
                ov = uid_v[pl.ds(c * CHUNK + r0, 16)]
                bv = bid_v[pl.ds(c * CHUNK + r0, 16)]
                for r in range(16):
                    uo = ov[r]
                    bo = bv[r]
                    u0 = ubuf_v[r0 + r, pl.ds(uo, 16)]
                    u1 = ubuf_v[r0 + r, pl.ds(uo + 16, 16)]
                    bb0 = bbuf_v[r0 + r, pl.ds(bo, 16)]
                    bb1 = bbuf_v[r0 + r, pl.ds(bo + 16, 16)]
                    p = u0 * bb0 * w0 + u1 * bb1 * w1
                    plsc.store_scatter(tr_v, [col_base + r], p)
                acc = jnp.full((16,), fcb, dtype=jnp.float32)
                for d in range(16):
                    acc = acc + tr_v[pl.ds(d * 17, 16)]
                out_v[pl.ds(c * CHUNK + r0, 16)] = acc
                return carry

            lax.fori_loop(0, CHUNK // 16, group_body, 0)

        pltpu.sync_copy(out_v, out_hbm.at[pl.ds(base, b_per_w)])

    return kfn


def _tail_T(table, q_full):
    tail = table[q_full * 128:]
    return jnp.pad(tail, ((0, 128 - tail.shape[0]), (0, 0))).T


def kernel(user_id, book_id, user_table, book_table, fc_w, fc_b):
    B = user_id.shape[0]
    nu = user_table.shape[0]
    nb = book_table.shape[0]
    qu = nu // 128
    qb = nb // 128
    u128, b128 = _build_detile(nu, nb)(
        user_table.T, book_table.T,
        _tail_T(user_table, qu), _tail_T(book_table, qb))
    w = fc_w.reshape(EMBED_DIM)
    b = jnp.pad(fc_b, (0, 15))
    return _build_gather(B)(user_id.astype(jnp.int32),
                            book_id.astype(jnp.int32), u128, b128, w, b)


# speedup vs baseline: 1.1137x; 1.1137x over previous
import functools

import jax
import jax.numpy as jnp
from jax import lax
from jax.experimental import pallas as pl
from jax.experimental.pallas import tpu as pltpu
from jax.experimental.pallas import tpu_sc as plsc

EMBED_DIM = 32
CHUNK = 128


@functools.lru_cache(maxsize=None)
def _build_detile(nu: int, nb: int):
    info = plsc.get_sparse_core_info()
    NC, NS = info.num_cores, info.num_subcores
    NW = NC * NS
    qu = nu // 128
    qb = nb // 128
    u_rows = (qu + 1) * 32
    b_rows = (qb + 1) * 32

    mesh = plsc.VectorSubcoreMesh(core_axis_name="c", subcore_axis_name="s")

    @functools.partial(
        pl.kernel,
        mesh=mesh,
        compiler_params=pltpu.CompilerParams(needs_layout_passes=False),
        out_type=(jax.ShapeDtypeStruct((u_rows, 128), jnp.float32),
                  jax.ShapeDtypeStruct((b_rows, 128), jnp.float32)),
        scratch_types=[
            pltpu.VMEM((EMBED_DIM, 129), jnp.float32),
            pltpu.VMEM((EMBED_DIM, 129), jnp.float32),
            pltpu.VMEM((EMBED_DIM, 128), jnp.float32),
            pltpu.VMEM((EMBED_DIM, 128), jnp.float32),
            pltpu.SemaphoreType.DMA,
            pltpu.SemaphoreType.DMA,
            pltpu.SemaphoreType.DMA,
        ],
    )
    def kfn(utabT, btabT, utailT, btailT, u128, b128,
            cvmA, cvmB, tvmA, tvmB, ldsem, stsemA, stsemB):
        wid = lax.axis_index("s") * NC + lax.axis_index("c")
        iota16 = lax.iota(jnp.int32, 16)
        iota16b = iota16 + 16

        def issue(src, src_q, cvm):
            col0 = pl.multiple_of(src_q * 128, 128)
            return [pltpu.async_copy(
                src.at[pl.ds(8 * p, 8), pl.ds(col0, 128)],
                cvm.at[pl.ds(8 * p, 8), pl.ds(0, 128)], ldsem)
                for p in range(4)]

        def transpose(cvm, tvm):
            def tr_body(rr, carry):
                rv = jnp.broadcast_to(rr, (16,)).astype(jnp.int32)
                g0 = plsc.load_gather(cvm, [iota16, rv])
                g1 = plsc.load_gather(cvm, [iota16b, rv])
                row = rr // 4
                col = (rr % 4) * 32
                tvm[row, pl.ds(col, 16)] = g0
                tvm[row, pl.ds(col + 16, 16)] = g1
                return carry

            lax.fori_loop(0, 128, tr_body, 0, unroll=8)

        def wait_write(tvm, stsem):
            pltpu.make_async_copy(tvm, u128.at[pl.ds(0, 32)], stsem).wait()

        def wait_loads(src):
            for p in range(4):
                pltpu.make_async_copy(
                    src.at[pl.ds(0, 8), pl.ds(0, 128)],
                    cvmA.at[pl.ds(0, 8), pl.ds(0, 128)], ldsem).wait()

        def make_pair_body(src, dst, limit, first_user):
            def pair_body(k, carry):
                qA = wid + (2 * k) * NW
                qB = wid + (2 * k + 1) * NW

                @pl.when(qA < limit)
                def _():
                    issue(src, qA, cvmA)

                @pl.when(qB < limit)
                def _():
                    issue(src, qB, cvmB)

                @pl.when(qA < limit)
                def _():
                    if first_user:
                        @pl.when(k > 0)
                        def _():
                            wait_write(tvmA, stsemA)
                    else:
                        wait_write(tvmA, stsemA)
                    wait_loads(src)
                    transpose(cvmA, tvmA)
                    row0 = pl.multiple_of(qA * 32, 32)
                    pltpu.async_copy(tvmA, dst.at[pl.ds(row0, 32)], stsemA)

                @pl.when(qB < limit)
                def _():
                    if first_user:
                        @pl.when(k > 0)
                        def _():
                            wait_write(tvmB, stsemB)
                    else:
                        wait_write(tvmB, stsemB)
                    wait_loads(src)
                    transpose(cvmB, tvmB)
                    row0 = pl.multiple_of(qB * 32, 32)
                    pltpu.async_copy(tvmB, dst.at[pl.ds(row0, 32)], stsemB)

                return carry
            return pair_body

        lax.fori_loop(0, (qu + 2 * NW - 1) // (2 * NW),
                      make_pair_body(utabT, u128, qu, True), 0)
        lax.fori_loop(0, (qb + 2 * NW - 1) // (2 * NW),
                      make_pair_body(btabT, b128, qb, False), 0)

        @pl.when(wid == 0)
        def _():
            loads = issue(utailT, 0, cvmA)
            wait_write(tvmA, stsemA)
            for cc in loads:
                cc.wait()
            transpose(cvmA, tvmA)
            pltpu.async_copy(tvmA, u128.at[pl.ds(qu * 32, 32)], stsemA)

        @pl.when(wid == 1)
        def _():
            loads = issue(btailT, 0, cvmB)
            wait_write(tvmB, stsemB)
            for cc in loads:
                cc.wait()
            transpose(cvmB, tvmB)
            pltpu.async_copy(tvmB, b128.at[pl.ds(qb * 32, 32)], stsemB)

        wait_write(tvmA, stsemA)
        wait_write(tvmB, stsemB)

    return kfn


@functools.lru_cache(maxsize=None)
def _build_gather(B: int):
    info = plsc.get_sparse_core_info()
    NC, NS = info.num_cores, info.num_subcores
    NW = NC * NS
    b_per_w = B // NW
    n_chunks = b_per_w // CHUNK

    mesh = plsc.VectorSubcoreMesh(core_axis_name="c", subcore_axis_name="s")

    @functools.partial(
        pl.kernel,
        mesh=mesh,
        compiler_params=pltpu.CompilerParams(needs_layout_passes=False),
        out_type=jax.ShapeDtypeStruct((B,), jnp.float32),
        scratch_types=[
            pltpu.VMEM((b_per_w,), jnp.int32),
            pltpu.VMEM((b_per_w,), jnp.int32),
            pltpu.VMEM((b_per_w,), jnp.int32),
            pltpu.VMEM((b_per_w,), jnp.int32),
            pltpu.VMEM((CHUNK, 128), jnp.float32),
            pltpu.VMEM((CHUNK, 128), jnp.float32),
            pltpu.VMEM((EMBED_DIM,), jnp.float32),
            pltpu.VMEM((16,), jnp.float32),
            pltpu.VMEM((b_per_w,), jnp.float32),
            pltpu.VMEM((17 * 16,), jnp.float32),
            pltpu.SemaphoreType.DMA,
        ],
    )
    def kfn(uid_hbm, bid_hbm, utab_hbm, btab_hbm, w_hbm, b_hbm, out_hbm,
            uid_v, bid_v, uq_v, bq_v, ubuf_v, bbuf_v, w_v, b_v, out_v, tr_v,
            gsem):
        wid = lax.axis_index("s") * NC + lax.axis_index("c")
        base = wid * b_per_w

        pltpu.sync_copy(uid_hbm.at[pl.ds(base, b_per_w)], uid_v)
        pltpu.sync_copy(bid_hbm.at[pl.ds(base, b_per_w)], bid_v)
        pltpu.sync_copy(w_hbm, w_v)
        pltpu.sync_copy(b_hbm, b_v)

        def split_body(g, carry):
            sl = pl.ds(g * 16, 16)
            for src, qdst in ((uid_v, uq_v), (bid_v, bq_v)):
                v = src[sl]
                qdst[sl] = lax.shift_right_logical(v, 2)
                src[sl] = lax.shift_left(v & 3, 5)
            return carry

        lax.fori_loop(0, b_per_w // 16, split_body, 0)

        w0 = w_v[pl.ds(0, 16)]
        w1 = w_v[pl.ds(16, 16)]
        fcb = b_v[pl.ds(0, 16)][0]
        col_base = lax.iota(jnp.int32, 16) * 17

        for c in range(n_chunks):
            sl = pl.ds(c * CHUNK, CHUNK)
            cu = pltpu.async_copy(utab_hbm.at[uq_v.at[sl]], ubuf_v, gsem)
            cb = pltpu.async_copy(btab_hbm.at[bq_v.at[sl]], bbuf_v, gsem)
            cu.wait()
            cb.wait()

            def group_body(g, carry, c=c):
                r0 = g * 16
                ov = uid_v[pl.ds(c * CHUNK + r0, 16)]
                bv = bid_v[pl.ds(c * CHUNK + r0, 16)]
                for r in range(16):
                    uo = ov[r]
                    bo = bv[r]
                    u0 = ubuf_v[r0 + r, pl.ds(uo, 16)]
                    u1 = ubuf_v[r0 + r, pl.ds(uo + 16, 16)]
                    bb0 = bbuf_v[r0 + r, pl.ds(bo, 16)]
                    bb1 = bbuf_v[r0 + r, pl.ds(bo + 16, 16)]
                    p = u0 * bb0 * w0 + u1 * bb1 * w1
                    plsc.store_scatter(tr_v, [col_base + r], p)
                acc = jnp.full((16,), fcb, dtype=jnp.float32)
                for d in range(16):
                    acc = acc + tr_v[pl.ds(d * 17, 16)]
                out_v[pl.ds(c * CHUNK + r0, 16)] = acc
                return carry

            lax.fori_loop(0, CHUNK // 16, group_body, 0)

        pltpu.sync_copy(out_v, out_hbm.at[pl.ds(base, b_per_w)])

    return kfn


def _tail_T(table, q_full):
    tail = table[q_full * 128:]
    return jnp.pad(tail, ((0, 128 - tail.shape[0]), (0, 0))).T


def kernel(user_id, book_id, user_table, book_table, fc_w, fc_b):
    B = user_id.shape[0]
    nu = user_table.shape[0]
    nb = book_table.shape[0]
    qu = nu // 128
    qb = nb // 128
    u128, b128 = _build_detile(nu, nb)(
        user_table.T, book_table.T,
        _tail_T(user_table, qu), _tail_T(book_table, qb))
    w = fc_w.reshape(EMBED_DIM)
    b = jnp.pad(fc_b, (0, 15))
    return _build_gather(B)(user_id.astype(jnp.int32),
                            book_id.astype(jnp.int32), u128, b128, w, b)


# speedup vs baseline: 1.4769x; 1.3260x over previous
import functools

import jax
import jax.numpy as jnp
from jax import lax
from jax.experimental import pallas as pl
from jax.experimental.pallas import tpu as pltpu
from jax.experimental.pallas import tpu_sc as plsc

EMBED_DIM = 32
CHUNK = 128


@functools.lru_cache(maxsize=None)
def _build_detile(nu: int, nb: int):
    info = plsc.get_sparse_core_info()
    NC, NS = info.num_cores, info.num_subcores
    NW = NC * NS
    qu = nu // 128
    qb = nb // 128
    u_rows = (qu + 1) * 32
    b_rows = (qb + 1) * 32

    mesh = plsc.VectorSubcoreMesh(core_axis_name="c", subcore_axis_name="s")

    @functools.partial(
        pl.kernel,
        mesh=mesh,
        compiler_params=pltpu.CompilerParams(needs_layout_passes=False),
        out_type=(jax.ShapeDtypeStruct((u_rows, 128), jnp.float32),
                  jax.ShapeDtypeStruct((b_rows, 128), jnp.float32)),
        scratch_types=[
            pltpu.VMEM((EMBED_DIM, 129), jnp.float32),
            pltpu.VMEM((EMBED_DIM, 129), jnp.float32),
            pltpu.VMEM((EMBED_DIM, 128), jnp.float32),
            pltpu.VMEM((EMBED_DIM, 128), jnp.float32),
            pltpu.SemaphoreType.DMA,
            pltpu.SemaphoreType.DMA,
            pltpu.SemaphoreType.DMA,
        ],
    )
    def kfn(utabT, btabT, utailT, btailT, u128, b128,
            cvmA, cvmB, tvmA, tvmB, ldsem, stsemA, stsemB):
        wid = lax.axis_index("s") * NC + lax.axis_index("c")
        iota16 = lax.iota(jnp.int32, 16)
        iota16b = iota16 + 16

        def issue(src, src_q, cvm):
            col0 = pl.multiple_of(src_q * 128, 128)
            return [pltpu.async_copy(
                src.at[pl.ds(8 * p, 8), pl.ds(col0, 128)],
                cvm.at[pl.ds(8 * p, 8), pl.ds(0, 128)], ldsem)
                for p in range(4)]

        def transpose(cvm, tvm):
            @plsc.parallel_loop(0, 128, unroll=8)
            def tr_body(rr):
                rv = jnp.broadcast_to(rr, (16,)).astype(jnp.int32)
                g0 = plsc.load_gather(cvm, [iota16, rv])
                g1 = plsc.load_gather(cvm, [iota16b, rv])
                row = rr // 4
                col = (rr % 4) * 32
                tvm[row, pl.ds(col, 16)] = g0
                tvm[row, pl.ds(col + 16, 16)] = g1

        def wait_write(tvm, stsem):
            pltpu.make_async_copy(tvm, u128.at[pl.ds(0, 32)], stsem).wait()

        def wait_loads(src):
            for p in range(4):
                pltpu.make_async_copy(
                    src.at[pl.ds(0, 8), pl.ds(0, 128)],
                    cvmA.at[pl.ds(0, 8), pl.ds(0, 128)], ldsem).wait()

        def make_pair_body(src, dst, limit, first_user):
            def pair_body(k, carry):
                qA = wid + (2 * k) * NW
                qB = wid + (2 * k + 1) * NW

                @pl.when(qA < limit)
                def _():
                    issue(src, qA, cvmA)

                @pl.when(qB < limit)
                def _():
                    issue(src, qB, cvmB)

                @pl.when(qA < limit)
                def _():
                    if first_user:
                        @pl.when(k > 0)
                        def _():
                            wait_write(tvmA, stsemA)
                    else:
                        wait_write(tvmA, stsemA)
                    wait_loads(src)
                    transpose(cvmA, tvmA)
                    row0 = pl.multiple_of(qA * 32, 32)
                    pltpu.async_copy(tvmA, dst.at[pl.ds(row0, 32)], stsemA)

                @pl.when(qB < limit)
                def _():
                    if first_user:
                        @pl.when(k > 0)
                        def _():
                            wait_write(tvmB, stsemB)
                    else:
                        wait_write(tvmB, stsemB)
                    wait_loads(src)
                    transpose(cvmB, tvmB)
                    row0 = pl.multiple_of(qB * 32, 32)
                    pltpu.async_copy(tvmB, dst.at[pl.ds(row0, 32)], stsemB)

                return carry
            return pair_body

        lax.fori_loop(0, (qu + 2 * NW - 1) // (2 * NW),
                      make_pair_body(utabT, u128, qu, True), 0)
        lax.fori_loop(0, (qb + 2 * NW - 1) // (2 * NW),
                      make_pair_body(btabT, b128, qb, False), 0)

        @pl.when(wid == 0)
        def _():
            loads = issue(utailT, 0, cvmA)
            wait_write(tvmA, stsemA)
            for cc in loads:
                cc.wait()
            transpose(cvmA, tvmA)
            pltpu.async_copy(tvmA, u128.at[pl.ds(qu * 32, 32)], stsemA)

        @pl.when(wid == 1)
        def _():
            loads = issue(btailT, 0, cvmB)
            wait_write(tvmB, stsemB)
            for cc in loads:
                cc.wait()
            transpose(cvmB, tvmB)
            pltpu.async_copy(tvmB, b128.at[pl.ds(qb * 32, 32)], stsemB)

        wait_write(tvmA, stsemA)
        wait_write(tvmB, stsemB)

    return kfn


@functools.lru_cache(maxsize=None)
def _build_gather(B: int):
    info = plsc.get_sparse_core_info()
    NC, NS = info.num_cores, info.num_subcores
    NW = NC * NS
    b_per_w = B // NW
    n_chunks = b_per_w // CHUNK

    mesh = plsc.VectorSubcoreMesh(core_axis_name="c", subcore_axis_name="s")

    @functools.partial(
        pl.kernel,
        mesh=mesh,
        compiler_params=pltpu.CompilerParams(needs_layout_passes=False),
        out_type=jax.ShapeDtypeStruct((B,), jnp.float32),
        scratch_types=[
            pltpu.VMEM((b_per_w,), jnp.int32),
            pltpu.VMEM((b_per_w,), jnp.int32),
            pltpu.VMEM((b_per_w,), jnp.int32),
            pltpu.VMEM((b_per_w,), jnp.int32),
            pltpu.VMEM((CHUNK, 128), jnp.float32),
            pltpu.VMEM((CHUNK, 128), jnp.float32),
            pltpu.VMEM((EMBED_DIM,), jnp.float32),
            pltpu.VMEM((16,), jnp.float32),
            pltpu.VMEM((b_per_w,), jnp.float32),
            pltpu.VMEM((17 * 16,), jnp.float32),
            pltpu.SemaphoreType.DMA,
        ],
    )
    def kfn(uid_hbm, bid_hbm, utab_hbm, btab_hbm, w_hbm, b_hbm, out_hbm,
            uid_v, bid_v, uq_v, bq_v, ubuf_v, bbuf_v, w_v, b_v, out_v, tr_v,
            gsem):
        wid = lax.axis_index("s") * NC + lax.axis_index("c")
        base = wid * b_per_w

        pltpu.sync_copy(uid_hbm.at[pl.ds(base, b_per_w)], uid_v)
        pltpu.sync_copy(bid_hbm.at[pl.ds(base, b_per_w)], bid_v)
        pltpu.sync_copy(w_hbm, w_v)
        pltpu.sync_copy(b_hbm, b_v)

        def split_body(g, carry):
            sl = pl.ds(g * 16, 16)
            for src, qdst in ((uid_v, uq_v), (bid_v, bq_v)):
                v = src[sl]
                qdst[sl] = lax.shift_right_logical(v, 2)
                src[sl] = lax.shift_left(v & 3, 5)
            return carry

        lax.fori_loop(0, b_per_w // 16, split_body, 0)

        w0 = w_v[pl.ds(0, 16)]
        w1 = w_v[pl.ds(16, 16)]
        fcb = b_v[pl.ds(0, 16)][0]
        col_base = lax.iota(jnp.int32, 16) * 17

        for c in range(n_chunks):
            sl = pl.ds(c * CHUNK, CHUNK)
            cu = pltpu.async_copy(utab_hbm.at[uq_v.at[sl]], ubuf_v, gsem)
            cb = pltpu.async_copy(btab_hbm.at[bq_v.at[sl]], bbuf_v, gsem)
            cu.wait()
            cb.wait()

            def group_body(g, carry, c=c):
                r0 = g * 16
                ov = uid_v[pl.ds(c * CHUNK + r0, 16)]
                bv = bid_v[pl.ds(c * CHUNK + r0, 16)]
                for r in range(16):
                    uo = ov[r]
                    bo = bv[r]
                    u0 = ubuf_v[r0 + r, pl.ds(uo, 16)]
                    u1 = ubuf_v[r0 + r, pl.ds(uo + 16, 16)]
                    bb0 = bbuf_v[r0 + r, pl.ds(bo, 16)]
                    bb1 = bbuf_v[r0 + r, pl.ds(bo + 16, 16)]
                    p = u0 * bb0 * w0 + u1 * bb1 * w1
                    plsc.store_scatter(tr_v, [col_base + r], p)
                acc = jnp.full((16,), fcb, dtype=jnp.float32)
                for d in range(16):
                    acc = acc + tr_v[pl.ds(d * 17, 16)]
                out_v[pl.ds(c * CHUNK + r0, 16)] = acc
                return carry

            lax.fori_loop(0, CHUNK // 16, group_body, 0)

        pltpu.sync_copy(out_v, out_hbm.at[pl.ds(base, b_per_w)])

    return kfn


def _tail_T(table, q_full):
    tail = table[q_full * 128:]
    return jnp.pad(tail, ((0, 128 - tail.shape[0]), (0, 0))).T


def kernel(user_id, book_id, user_table, book_table, fc_w, fc_b):
    B = user_id.shape[0]
    nu = user_table.shape[0]
    nb = book_table.shape[0]
    qu = nu // 128
    qb = nb // 128
    u128, b128 = _build_detile(nu, nb)(
        user_table.T, book_table.T,
        _tail_T(user_table, qu), _tail_T(book_table, qb))
    w = fc_w.reshape(EMBED_DIM)
    b = jnp.pad(fc_b, (0, 15))
    return _build_gather(B)(user_id.astype(jnp.int32),
                            book_id.astype(jnp.int32), u128, b128, w, b)


# speedup vs baseline: 1.7709x; 1.1991x over previous
import functools

import jax
import jax.numpy as jnp
from jax import lax
from jax.experimental import pallas as pl
from jax.experimental.pallas import tpu as pltpu
from jax.experimental.pallas import tpu_sc as plsc

EMBED_DIM = 32
CHUNK = 128


@functools.lru_cache(maxsize=None)
def _build_detile(nu: int, nb: int):
    info = plsc.get_sparse_core_info()
    NC, NS = info.num_cores, info.num_subcores
    NW = NC * NS
    qu = nu // 128
    qb = nb // 128
    u_rows = (qu + 1) * 32
    b_rows = (qb + 1) * 32

    mesh = plsc.VectorSubcoreMesh(core_axis_name="c", subcore_axis_name="s")

    @functools.partial(
        pl.kernel,
        mesh=mesh,
        compiler_params=pltpu.CompilerParams(needs_layout_passes=False),
        out_type=(jax.ShapeDtypeStruct((u_rows, 128), jnp.float32),
                  jax.ShapeDtypeStruct((b_rows, 128), jnp.float32)),
        scratch_types=[
            pltpu.VMEM((EMBED_DIM, 129), jnp.float32),
            pltpu.VMEM((EMBED_DIM, 129), jnp.float32),
            pltpu.VMEM((EMBED_DIM, 128), jnp.float32),
            pltpu.VMEM((EMBED_DIM, 128), jnp.float32),
            pltpu.SemaphoreType.DMA,
            pltpu.SemaphoreType.DMA,
            pltpu.SemaphoreType.DMA,
        ],
    )
    def kfn(utabT, btabT, utailT, btailT, u128, b128,
            cvmA, cvmB, tvmA, tvmB, ldsem, stsemA, stsemB):
        wid = lax.axis_index("s") * NC + lax.axis_index("c")
        iota16 = lax.iota(jnp.int32, 16)
        iota16b = iota16 + 16

        def issue(src, src_q, cvm):
            col0 = pl.multiple_of(src_q * 128, 128)
            return [pltpu.async_copy(
                src.at[pl.ds(8 * p, 8), pl.ds(col0, 128)],
                cvm.at[pl.ds(8 * p, 8), pl.ds(0, 128)], ldsem)
                for p in range(4)]

        def transpose(cvm, tvm):
            @plsc.parallel_loop(0, 128, unroll=8)
            def tr_body(rr):
                rv = jnp.broadcast_to(rr, (16,)).astype(jnp.int32)
                g0 = plsc.load_gather(cvm, [iota16, rv])
                g1 = plsc.load_gather(cvm, [iota16b, rv])
                row = rr // 4
                col = (rr % 4) * 32
                tvm[row, pl.ds(col, 16)] = g0
                tvm[row, pl.ds(col + 16, 16)] = g1

        def wait_write(tvm, stsem):
            pltpu.make_async_copy(tvm, u128.at[pl.ds(0, 32)], stsem).wait()

        def wait_loads(src):
            for p in range(4):
                pltpu.make_async_copy(
                    src.at[pl.ds(0, 8), pl.ds(0, 128)],
                    cvmA.at[pl.ds(0, 8), pl.ds(0, 128)], ldsem).wait()

        def make_pair_body(src, dst, limit, first_user):
            def pair_body(k, carry):
                qA = wid + (2 * k) * NW
                qB = wid + (2 * k + 1) * NW

                @pl.when(qA < limit)
                def _():
                    if first_user:
                        @pl.when(k > 0)
                        def _():
                            wait_write(tvmA, stsemA)
                    else:
                        wait_write(tvmA, stsemA)
                    wait_loads(src)
                    transpose(cvmA, tvmA)
                    row0 = pl.multiple_of(qA * 32, 32)
                    pltpu.async_copy(tvmA, dst.at[pl.ds(row0, 32)], stsemA)

                @pl.when(qA + 2 * NW < limit)
                def _():
                    issue(src, qA + 2 * NW, cvmA)

                @pl.when(qB < limit)
                def _():
                    if first_user:
                        @pl.when(k > 0)
                        def _():
                            wait_write(tvmB, stsemB)
                    else:
                        wait_write(tvmB, stsemB)
                    wait_loads(src)
                    transpose(cvmB, tvmB)
                    row0 = pl.multiple_of(qB * 32, 32)
                    pltpu.async_copy(tvmB, dst.at[pl.ds(row0, 32)], stsemB)

                @pl.when(qB + 2 * NW < limit)
                def _():
                    issue(src, qB + 2 * NW, cvmB)

                return carry
            return pair_body

        @pl.when(wid < qu)
        def _():
            issue(utabT, wid, cvmA)

        @pl.when(wid + NW < qu)
        def _():
            issue(utabT, wid + NW, cvmB)

        lax.fori_loop(0, (qu + 2 * NW - 1) // (2 * NW),
                      make_pair_body(utabT, u128, qu, True), 0)

        @pl.when(wid < qb)
        def _():
            issue(btabT, wid, cvmA)

        @pl.when(wid + NW < qb)
        def _():
            issue(btabT, wid + NW, cvmB)

        lax.fori_loop(0, (qb + 2 * NW - 1) // (2 * NW),
                      make_pair_body(btabT, b128, qb, False), 0)

        @pl.when(wid == 0)
        def _():
            loads = issue(utailT, 0, cvmA)
            wait_write(tvmA, stsemA)
            for cc in loads:
                cc.wait()
            transpose(cvmA, tvmA)
            pltpu.async_copy(tvmA, u128.at[pl.ds(qu * 32, 32)], stsemA)

        @pl.when(wid == 1)
        def _():
            loads = issue(btailT, 0, cvmB)
            wait_write(tvmB, stsemB)
            for cc in loads:
                cc.wait()
            transpose(cvmB, tvmB)
            pltpu.async_copy(tvmB, b128.at[pl.ds(qb * 32, 32)], stsemB)

        wait_write(tvmA, stsemA)
        wait_write(tvmB, stsemB)

    return kfn


@functools.lru_cache(maxsize=None)
def _build_gather(B: int):
    info = plsc.get_sparse_core_info()
    NC, NS = info.num_cores, info.num_subcores
    NW = NC * NS
    b_per_w = B // NW
    n_chunks = b_per_w // CHUNK

    mesh = plsc.VectorSubcoreMesh(core_axis_name="c", subcore_axis_name="s")

    @functools.partial(
        pl.kernel,
        mesh=mesh,
        compiler_params=pltpu.CompilerParams(needs_layout_passes=False),
        out_type=jax.ShapeDtypeStruct((B,), jnp.float32),
        scratch_types=[
            pltpu.VMEM((b_per_w,), jnp.int32),
            pltpu.VMEM((b_per_w,), jnp.int32),
            pltpu.VMEM((b_per_w,), jnp.int32),
            pltpu.VMEM((b_per_w,), jnp.int32),
            pltpu.VMEM((CHUNK, 128), jnp.float32),
            pltpu.VMEM((CHUNK, 128), jnp.float32),
            pltpu.VMEM((EMBED_DIM,), jnp.float32),
            pltpu.VMEM((16,), jnp.float32),
            pltpu.VMEM((b_per_w,), jnp.float32),
            pltpu.VMEM((17 * 16,), jnp.float32),
            pltpu.SemaphoreType.DMA,
        ],
    )
    def kfn(uid_hbm, bid_hbm, utab_hbm, btab_hbm, w_hbm, b_hbm, out_hbm,
            uid_v, bid_v, uq_v, bq_v, ubuf_v, bbuf_v, w_v, b_v, out_v, tr_v,
            gsem):
        wid = lax.axis_index("s") * NC + lax.axis_index("c")
        base = wid * b_per_w

        pltpu.sync_copy(uid_hbm.at[pl.ds(base, b_per_w)], uid_v)
        pltpu.sync_copy(bid_hbm.at[pl.ds(base, b_per_w)], bid_v)
        pltpu.sync_copy(w_hbm, w_v)
        pltpu.sync_copy(b_hbm, b_v)

        def split_body(g, carry):
            sl = pl.ds(g * 16, 16)
            for src, qdst in ((uid_v, uq_v), (bid_v, bq_v)):
                v = src[sl]
                qdst[sl] = lax.shift_right_logical(v, 2)
                src[sl] = lax.shift_left(v & 3, 5)
            return carry

        lax.fori_loop(0, b_per_w // 16, split_body, 0)

        w0 = w_v[pl.ds(0, 16)]
        w1 = w_v[pl.ds(16, 16)]
        fcb = b_v[pl.ds(0, 16)][0]
        col_base = lax.iota(jnp.int32, 16) * 17

        for c in range(n_chunks):
            sl = pl.ds(c * CHUNK, CHUNK)
            cu = pltpu.async_copy(utab_hbm.at[uq_v.at[sl]], ubuf_v, gsem)
            cb = pltpu.async_copy(btab_hbm.at[bq_v.at[sl]], bbuf_v, gsem)
            cu.wait()
            cb.wait()

            def group_body(g, carry, c=c):
                r0 = g * 16
                ov = uid_v[pl.ds(c * CHUNK + r0, 16)]
                bv = bid_v[pl.ds(c * CHUNK + r0, 16)]
                for r in range(16):
                    uo = ov[r]
                    bo = bv[r]
                    u0 = ubuf_v[r0 + r, pl.ds(uo, 16)]
                    u1 = ubuf_v[r0 + r, pl.ds(uo + 16, 16)]
                    bb0 = bbuf_v[r0 + r, pl.ds(bo, 16)]
                    bb1 = bbuf_v[r0 + r, pl.ds(bo + 16, 16)]
                    p = u0 * bb0 * w0 + u1 * bb1 * w1
                    plsc.store_scatter(tr_v, [col_base + r], p)
                acc = jnp.full((16,), fcb, dtype=jnp.float32)
                for d in range(16):
                    acc = acc + tr_v[pl.ds(d * 17, 16)]
                out_v[pl.ds(c * CHUNK + r0, 16)] = acc
                return carry

            lax.fori_loop(0, CHUNK // 16, group_body, 0)

        pltpu.sync_copy(out_v, out_hbm.at[pl.ds(base, b_per_w)])

    return kfn


def _tail_T(table, q_full):
    tail = table[q_full * 128:]
    return jnp.pad(tail, ((0, 128 - tail.shape[0]), (0, 0))).T


def kernel(user_id, book_id, user_table, book_table, fc_w, fc_b):
    B = user_id.shape[0]
    nu = user_table.shape[0]
    nb = book_table.shape[0]
    qu = nu // 128
    qb = nb // 128
    u128, b128 = _build_detile(nu, nb)(
        user_table.T, book_table.T,
        _tail_T(user_table, qu), _tail_T(book_table, qb))
    w = fc_w.reshape(EMBED_DIM)
    b = jnp.pad(fc_b, (0, 15))
    return _build_gather(B)(user_id.astype(jnp.int32),
                            book_id.astype(jnp.int32), u128, b128, w, b)


# speedup vs baseline: 1.8310x; 1.0339x over previous
import functools

import jax
import jax.numpy as jnp
from jax import lax
from jax.experimental import pallas as pl
from jax.experimental.pallas import tpu as pltpu
from jax.experimental.pallas import tpu_sc as plsc

EMBED_DIM = 32
CHUNK = 128


@functools.lru_cache(maxsize=None)
def _build(B: int):
    info = plsc.get_sparse_core_info()
    NC, NS = info.num_cores, info.num_subcores
    NW = NC * NS
    b_per_w = B // NW
    n_chunks = b_per_w // CHUNK

    mesh = plsc.VectorSubcoreMesh(core_axis_name="c", subcore_axis_name="s")

    @functools.partial(
        pl.kernel,
        mesh=mesh,
        compiler_params=pltpu.CompilerParams(
            needs_layout_passes=False, use_tc_tiling_on_sc=False),
        out_type=jax.ShapeDtypeStruct((B,), jnp.float32),
        scratch_types=[
            pltpu.VMEM((n_chunks, CHUNK), jnp.int32),
            pltpu.VMEM((n_chunks, CHUNK), jnp.int32),
            pltpu.VMEM((b_per_w, EMBED_DIM), jnp.float32),
            pltpu.VMEM((b_per_w, EMBED_DIM), jnp.float32),
            pltpu.VMEM((EMBED_DIM,), jnp.float32),
            pltpu.VMEM((16,), jnp.float32),
            pltpu.VMEM((b_per_w,), jnp.float32),
            pltpu.VMEM((17 * 16,), jnp.float32),
            pltpu.SemaphoreType.DMA,
        ],
    )
    def kfn(uid_hbm, bid_hbm, utab_hbm, btab_hbm, w_hbm, b_hbm, out_hbm,
            uidx_v, bidx_v, urows_v, brows_v, w_v, b_v, out_v, tr_v, gsem):
        wid = lax.axis_index("s") * NC + lax.axis_index("c")
        base_row = wid * n_chunks

        pltpu.sync_copy(uid_hbm.at[pl.ds(base_row, n_chunks)], uidx_v)
        pltpu.sync_copy(bid_hbm.at[pl.ds(base_row, n_chunks)], bidx_v)
        pltpu.sync_copy(w_hbm, w_v)
        pltpu.sync_copy(b_hbm, b_v)

        copies = []
        for j in range(n_chunks):
            copies.append(pltpu.async_copy(
                utab_hbm.at[uidx_v.at[j]], urows_v.at[pl.ds(j * CHUNK, CHUNK)], gsem))
            copies.append(pltpu.async_copy(
                btab_hbm.at[bidx_v.at[j]], brows_v.at[pl.ds(j * CHUNK, CHUNK)], gsem))
        for c in copies:
            c.wait()

        w0 = w_v[pl.ds(0, 16)]
        w1 = w_v[pl.ds(16, 16)]
        fcb_vec = b_v[pl.ds(0, 16)]
        fcb = fcb_vec[0]
        col_base = lax.iota(jnp.int32, 16) * 17

        def group_body(g, carry):
            r0 = g * 16
            for r in range(16):
                u0 = urows_v[r0 + r, pl.ds(0, 16)]
                u1 = urows_v[r0 + r, pl.ds(16, 16)]
                bb0 = brows_v[r0 + r, pl.ds(0, 16)]
                bb1 = brows_v[r0 + r, pl.ds(16, 16)]
                p = u0 * bb0 * w0 + u1 * bb1 * w1
                plsc.store_scatter(tr_v, [col_base + r], p)
            acc = jnp.full((16,), fcb, dtype=jnp.float32)
            for d in range(16):
                acc = acc + tr_v[pl.ds(d * 17, 16)]
            out_v[pl.ds(r0, 16)] = acc
            return carry

        lax.fori_loop(0, b_per_w // 16, group_body, 0)

        pltpu.sync_copy(out_v, out_hbm.at[pl.ds(wid * b_per_w, b_per_w)])

    return kfn


def kernel(user_id, book_id, user_table, book_table, fc_w, fc_b):
    B = user_id.shape[0]
    uid2d = user_id.astype(jnp.int32).reshape(B // CHUNK, CHUNK)
    bid2d = book_id.astype(jnp.int32).reshape(B // CHUNK, CHUNK)
    w = fc_w.reshape(EMBED_DIM)
    b = jnp.pad(fc_b, (0, 15))
    return _build(B)(uid2d, bid2d, user_table, book_table, w, b)
